# baseline (speedup 1.0000x reference)
import jax
import jax.numpy as jnp
from jax.experimental import pallas as pl
from jax.experimental.pallas import tpu as pltpu


def _pick_block(n):
    for b in (256, 200, 128, 80, 40, 16, 8):
        if n % b == 0:
            return b
    return n


def _gnn_body(msteps, bm, n, adj_ref, feat_ref, w_ref, gamma_ref, beta_ref,
              out_ref, support_ref, stats_ref):
    i = pl.program_id(0)

    @pl.when(i == 0)
    def _init():
        support_ref[...] = jnp.dot(feat_ref[...], w_ref[...],
                                   preferred_element_type=jnp.float32
                                   ).astype(jnp.bfloat16)
        stats_ref[...] = jnp.zeros_like(stats_ref)

    @pl.when(i < msteps)
    def _compute():
        z = jnp.dot(adj_ref[...].astype(jnp.bfloat16), support_ref[...],
                    preferred_element_type=jnp.float32)
        out_ref[pl.ds(i * bm, bm), :] = z
        stats_ref[0:1, :] += jnp.sum(z, axis=0, keepdims=True)
        stats_ref[1:2, :] += jnp.sum(z * z, axis=0, keepdims=True)

    @pl.when(i == msteps)
    def _normalize():
        mean = stats_ref[0:1, :] / n
        var = stats_ref[1:2, :] / n - mean * mean
        inv = jax.lax.rsqrt(var + 1e-5)
        scale = inv * gamma_ref[...]
        shift = beta_ref[...] - mean * scale

        def body(j, _):
            blk = out_ref[pl.ds(j * bm, bm), :]
            out_ref[pl.ds(j * bm, bm), :] = jnp.maximum(blk * scale + shift,
                                                        0.0)
            return 0

        jax.lax.fori_loop(0, msteps, body, 0)


def kernel(features, adj, weight, bias, gamma, beta):
    del bias
    n, in_dim = features.shape
    out_dim = weight.shape[1]
    bm = _pick_block(n)
    msteps = n // bm

    gamma2 = gamma.reshape(1, out_dim)
    beta2 = beta.reshape(1, out_dim)

    def body(adj_ref, feat_ref, w_ref, gamma_ref, beta_ref,
             out_ref, support_ref, stats_ref):
        _gnn_body(msteps, bm, n, adj_ref, feat_ref, w_ref, gamma_ref,
                  beta_ref, out_ref, support_ref, stats_ref)

    return pl.pallas_call(
        body,
        grid=(msteps + 1,),
        in_specs=[
            pl.BlockSpec((bm, n), lambda i: (jnp.minimum(i, msteps - 1), 0)),
            pl.BlockSpec((n, in_dim), lambda i: (0, 0)),
            pl.BlockSpec((in_dim, out_dim), lambda i: (0, 0)),
            pl.BlockSpec((1, out_dim), lambda i: (0, 0)),
            pl.BlockSpec((1, out_dim), lambda i: (0, 0)),
        ],
        out_specs=pl.BlockSpec((n, out_dim), lambda i: (0, 0)),
        out_shape=jax.ShapeDtypeStruct((n, out_dim), jnp.float32),
        scratch_shapes=[
            pltpu.VMEM((n, out_dim), jnp.bfloat16),
            pltpu.VMEM((8, out_dim), jnp.float32),
        ],
        compiler_params=pltpu.CompilerParams(
            dimension_semantics=("arbitrary",),
            vmem_limit_bytes=100 * 1024 * 1024,
        ),
    )(adj, features, weight, gamma2, beta2)


# speedup vs baseline: 1.0105x; 1.0105x over previous
import jax
import jax.numpy as jnp
from jax.experimental import pallas as pl
from jax.experimental.pallas import tpu as pltpu


def _pick_block(n):
    for b in (256, 200, 128, 80, 40, 16, 8):
        if n % b == 0:
            return b
    return n


def _gnn_body(msteps, bm, n, adj_ref, feat_ref, w_ref, gamma_ref, beta_ref,
              out_ref, support_ref, stats_ref):
    i = pl.program_id(0)

    @pl.when(i == 0)
    def _init():
        support_ref[...] = jnp.dot(feat_ref[...], w_ref[...],
                                   preferred_element_type=jnp.float32)
        stats_ref[...] = jnp.zeros_like(stats_ref)

    @pl.when(i < msteps)
    def _compute():
        z = jnp.dot(adj_ref[...], support_ref[...],
                    preferred_element_type=jnp.float32)
        out_ref[pl.ds(i * bm, bm), :] = z
        stats_ref[0:1, :] += jnp.sum(z, axis=0, keepdims=True)
        stats_ref[1:2, :] += jnp.sum(z * z, axis=0, keepdims=True)

    @pl.when(i == msteps)
    def _normalize():
        mean = stats_ref[0:1, :] / n
        var = stats_ref[1:2, :] / n - mean * mean
        inv = jax.lax.rsqrt(var + 1e-5)
        scale = inv * gamma_ref[...]
        shift = beta_ref[...] - mean * scale

        def body(j, _):
            blk = out_ref[pl.ds(j * bm, bm), :]
            out_ref[pl.ds(j * bm, bm), :] = jnp.maximum(blk * scale + shift,
                                                        0.0)
            return 0

        jax.lax.fori_loop(0, msteps, body, 0)


def kernel(features, adj, weight, bias, gamma, beta):
    del bias
    n, in_dim = features.shape
    out_dim = weight.shape[1]
    bm = _pick_block(n)
    msteps = n // bm

    gamma2 = gamma.reshape(1, out_dim)
    beta2 = beta.reshape(1, out_dim)

    def body(adj_ref, feat_ref, w_ref, gamma_ref, beta_ref,
             out_ref, support_ref, stats_ref):
        _gnn_body(msteps, bm, n, adj_ref, feat_ref, w_ref, gamma_ref,
                  beta_ref, out_ref, support_ref, stats_ref)

    return pl.pallas_call(
        body,
        grid=(msteps + 1,),
        in_specs=[
            pl.BlockSpec((bm, n), lambda i: (jnp.minimum(i, msteps - 1), 0)),
            pl.BlockSpec((n, in_dim), lambda i: (0, 0)),
            pl.BlockSpec((in_dim, out_dim), lambda i: (0, 0)),
            pl.BlockSpec((1, out_dim), lambda i: (0, 0)),
            pl.BlockSpec((1, out_dim), lambda i: (0, 0)),
        ],
        out_specs=pl.BlockSpec((n, out_dim), lambda i: (0, 0)),
        out_shape=jax.ShapeDtypeStruct((n, out_dim), jnp.float32),
        scratch_shapes=[
            pltpu.VMEM((n, out_dim), jnp.float32),
            pltpu.VMEM((8, out_dim), jnp.float32),
        ],
        compiler_params=pltpu.CompilerParams(
            dimension_semantics=("arbitrary",),
            vmem_limit_bytes=100 * 1024 * 1024,
        ),
    )(adj, features, weight, gamma2, beta2)


# speedup vs baseline: 1.0142x; 1.0037x over previous
import jax
import jax.numpy as jnp
from jax.experimental import pallas as pl
from jax.experimental.pallas import tpu as pltpu


def _pick_block(n):
    for b in (200, 128, 80, 40, 16, 8):
        if n % (2 * b) == 0:
            return b
    return n


def _gnn_body(msteps, bm, n, adja_ref, adjb_ref, feat_ref, w_ref, gamma_ref,
              beta_ref, out_ref, support_ref, stats_ref):
    i = pl.program_id(0)

    @pl.when(i == 0)
    def _init():
        support_ref[...] = jnp.dot(feat_ref[...], w_ref[...],
                                   preferred_element_type=jnp.float32)
        stats_ref[...] = jnp.zeros_like(stats_ref)

    @pl.when(i < msteps)
    def _compute():
        za = jnp.dot(adja_ref[...], support_ref[...],
                     preferred_element_type=jnp.float32)
        zb = jnp.dot(adjb_ref[...], support_ref[...],
                     preferred_element_type=jnp.float32)
        out_ref[pl.ds(2 * i * bm, bm), :] = za
        out_ref[pl.ds((2 * i + 1) * bm, bm), :] = zb
        stats_ref[0:1, :] += (jnp.sum(za, axis=0, keepdims=True) +
                              jnp.sum(zb, axis=0, keepdims=True))
        stats_ref[1:2, :] += (jnp.sum(za * za, axis=0, keepdims=True) +
                              jnp.sum(zb * zb, axis=0, keepdims=True))

    @pl.when(i == msteps)
    def _normalize():
        mean = stats_ref[0:1, :] / n
        var = stats_ref[1:2, :] / n - mean * mean
        inv = jax.lax.rsqrt(var + 1e-5)
        scale = inv * gamma_ref[...]
        shift = beta_ref[...] - mean * scale

        def body(j, _):
            blk = out_ref[pl.ds(j * bm, bm), :]
            out_ref[pl.ds(j * bm, bm), :] = jnp.maximum(blk * scale + shift,
                                                        0.0)
            return 0

        jax.lax.fori_loop(0, 2 * msteps, body, 0)


def kernel(features, adj, weight, bias, gamma, beta):
    del bias
    n, in_dim = features.shape
    out_dim = weight.shape[1]
    bm = _pick_block(n)
    msteps = n // (2 * bm)

    gamma2 = gamma.reshape(1, out_dim)
    beta2 = beta.reshape(1, out_dim)

    def body(adja_ref, adjb_ref, feat_ref, w_ref, gamma_ref, beta_ref,
             out_ref, support_ref, stats_ref):
        _gnn_body(msteps, bm, n, adja_ref, adjb_ref, feat_ref, w_ref,
                  gamma_ref, beta_ref, out_ref, support_ref, stats_ref)

    last = msteps - 1
    return pl.pallas_call(
        body,
        grid=(msteps + 1,),
        in_specs=[
            pl.BlockSpec((bm, n),
                         lambda i: (2 * jnp.minimum(i, last), 0)),
            pl.BlockSpec((bm, n),
                         lambda i: (2 * jnp.minimum(i, last) + 1, 0)),
            pl.BlockSpec((n, in_dim), lambda i: (0, 0)),
            pl.BlockSpec((in_dim, out_dim), lambda i: (0, 0)),
            pl.BlockSpec((1, out_dim), lambda i: (0, 0)),
            pl.BlockSpec((1, out_dim), lambda i: (0, 0)),
        ],
        out_specs=pl.BlockSpec((n, out_dim), lambda i: (0, 0)),
        out_shape=jax.ShapeDtypeStruct((n, out_dim), jnp.float32),
        scratch_shapes=[
            pltpu.VMEM((n, out_dim), jnp.float32),
            pltpu.VMEM((8, out_dim), jnp.float32),
        ],
        compiler_params=pltpu.CompilerParams(
            dimension_semantics=("arbitrary",),
            vmem_limit_bytes=110 * 1024 * 1024,
        ),
    )(adj, adj, features, weight, gamma2, beta2)
